# baseline (speedup 1.0000x reference)
import jax
import jax.numpy as jnp
from jax import lax
from jax.experimental import pallas as pl
from jax.experimental.pallas import tpu as pltpu
from jax.experimental.pallas import tpu_sc as plsc

_N = 32768
_E = 8
_K = 2
_D = 768
_TOK_BLK = 8192

_NC = 2
_NS = 16
_L = 16
_NW = _NC * _NS


def _logits_body(w_ref, x_ref, out_ref):
    out_ref[...] = lax.dot_general(
        w_ref[...], x_ref[...],
        dimension_numbers=(((1,), (1,)), ((), ())),
        preferred_element_type=jnp.float32,
    )


def _compute_logits(x, gate_w):
    return pl.pallas_call(
        _logits_body,
        grid=(_N // _TOK_BLK,),
        in_specs=[
            pl.BlockSpec((_E, _D), lambda i: (0, 0)),
            pl.BlockSpec((_TOK_BLK, _D), lambda i: (i, 0)),
        ],
        out_specs=pl.BlockSpec((_E, _TOK_BLK), lambda i: (0, i)),
        out_shape=jax.ShapeDtypeStruct((_E, _N), jnp.float32),
        compiler_params=pltpu.CompilerParams(
            dimension_semantics=("parallel",),
        ),
    )(gate_w, x)


def _route_chunk(vs, off, iota, ones_f, mask_v, w_v, idx_v):
    m1 = vs[0]
    i1 = jnp.zeros((_L,), jnp.int32)
    for e in range(1, _E):
        b = vs[e] > m1
        m1 = jnp.where(b, vs[e], m1)
        i1 = jnp.where(b, jnp.full((_L,), e, jnp.int32), i1)
    m2 = jnp.full((_L,), -jnp.inf, jnp.float32)
    i2 = jnp.zeros((_L,), jnp.int32)
    for e in range(_E):
        b = jnp.logical_and(vs[e] > m2, i1 != e)
        m2 = jnp.where(b, vs[e], m2)
        i2 = jnp.where(b, jnp.full((_L,), e, jnp.int32), i2)
    t = jnp.exp(m2 - m1)
    denom = 1.0 + t
    w1 = 1.0 / denom
    w2 = t / denom

    tok = off + iota
    zeros_f = jnp.zeros((_L,), jnp.float32)
    for r in range(_E):
        mask_v[pl.ds(off * _E + r * _L, _L)] = zeros_f
    plsc.store_scatter(mask_v, [tok * _E + i1], ones_f)
    plsc.store_scatter(mask_v, [tok * _E + i2], ones_f)
    plsc.store_scatter(w_v, [tok * _K], w1)
    plsc.store_scatter(w_v, [tok * _K + 1], w2)
    plsc.store_scatter(idx_v, [tok * _K], i1)
    plsc.store_scatter(idx_v, [tok * _K + 1], i2)


def _make_route(n_tok):
    tpw = n_tok // _NW
    chunks = tpw // _L

    def body(lg_hbm, mask_hbm, w_hbm, idx_hbm, lg_v, mask_v, w_v, idx_v):
        wid = lax.axis_index("s") * _NC + lax.axis_index("c")
        base = wid * tpw
        pltpu.sync_copy(lg_hbm.at[:, pl.ds(base, tpw)], lg_v)

        iota = lax.iota(jnp.int32, _L)
        ones_f = jnp.ones((_L,), jnp.float32)

        def chunk(j, carry):
            off = j * _L
            vs = [lg_v[e, pl.ds(off, _L)] for e in range(_E)]
            _route_chunk(vs, off, iota, ones_f, mask_v, w_v, idx_v)
            return carry

        lax.fori_loop(0, chunks, chunk, 0)

        pltpu.sync_copy(mask_v, mask_hbm.at[pl.ds(base * _E, tpw * _E)])
        pltpu.sync_copy(w_v, w_hbm.at[pl.ds(base * _K, tpw * _K)])
        pltpu.sync_copy(idx_v, idx_hbm.at[pl.ds(base * _K, tpw * _K)])

    mesh = plsc.VectorSubcoreMesh(core_axis_name="c", subcore_axis_name="s")
    return pl.kernel(
        body,
        mesh=mesh,
        compiler_params=pltpu.CompilerParams(needs_layout_passes=False),
        out_type=[
            jax.ShapeDtypeStruct((n_tok * _E,), jnp.float32),
            jax.ShapeDtypeStruct((n_tok * _K,), jnp.float32),
            jax.ShapeDtypeStruct((n_tok * _K,), jnp.int32),
        ],
        scratch_types=[
            pltpu.VMEM((_E, tpw), jnp.float32),
            pltpu.VMEM((tpw * _E,), jnp.float32),
            pltpu.VMEM((tpw * _K,), jnp.float32),
            pltpu.VMEM((tpw * _K,), jnp.int32),
        ],
    )


def kernel(x, gate_w):
    logits_t = _compute_logits(x, gate_w)
    mask, w, idx = _make_route(_N)(logits_t)
    return (
        mask.reshape(_N, _E),
        w.reshape(_N, _K),
        idx.reshape(_N, _K),
    )


# speedup vs baseline: 1.0205x; 1.0205x over previous
import jax
import jax.numpy as jnp
from jax import lax
from jax.experimental import pallas as pl
from jax.experimental.pallas import tpu as pltpu
from jax.experimental.pallas import tpu_sc as plsc

_N = 32768
_E = 8
_K = 2
_D = 768
_TOK_BLK = 4096

_NC = 2
_NS = 16
_L = 16
_NW = _NC * _NS


def _logits_body(w_ref, x_ref, out_ref):
    out_ref[...] = lax.dot_general(
        w_ref[...], x_ref[...],
        dimension_numbers=(((1,), (1,)), ((), ())),
        preferred_element_type=jnp.float32,
    )


def _compute_logits(x, gate_w):
    return pl.pallas_call(
        _logits_body,
        grid=(_N // _TOK_BLK,),
        in_specs=[
            pl.BlockSpec((_E, _D), lambda i: (0, 0)),
            pl.BlockSpec((_TOK_BLK, _D), lambda i: (i, 0)),
        ],
        out_specs=pl.BlockSpec((_E, _TOK_BLK), lambda i: (0, i)),
        out_shape=jax.ShapeDtypeStruct((_E, _N), jnp.float32),
        compiler_params=pltpu.CompilerParams(
            dimension_semantics=("parallel",),
        ),
    )(gate_w, x)


def _route_chunk(vs, off, iota, ones_f, mask_v, w_v, idx_v):
    m1 = vs[0]
    i1 = jnp.zeros((_L,), jnp.int32)
    for e in range(1, _E):
        b = vs[e] > m1
        m1 = jnp.where(b, vs[e], m1)
        i1 = jnp.where(b, jnp.full((_L,), e, jnp.int32), i1)
    m2 = jnp.full((_L,), -jnp.inf, jnp.float32)
    i2 = jnp.zeros((_L,), jnp.int32)
    for e in range(_E):
        b = jnp.logical_and(vs[e] > m2, i1 != e)
        m2 = jnp.where(b, vs[e], m2)
        i2 = jnp.where(b, jnp.full((_L,), e, jnp.int32), i2)
    t = jnp.exp(m2 - m1)
    denom = 1.0 + t
    w1 = 1.0 / denom
    w2 = t / denom

    tok = off + iota
    zeros_f = jnp.zeros((_L,), jnp.float32)
    for r in range(_E):
        mask_v[pl.ds(off * _E + r * _L, _L)] = zeros_f
    plsc.store_scatter(mask_v, [tok * _E + i1], ones_f)
    plsc.store_scatter(mask_v, [tok * _E + i2], ones_f)
    plsc.store_scatter(w_v, [tok * _K], w1)
    plsc.store_scatter(w_v, [tok * _K + 1], w2)
    plsc.store_scatter(idx_v, [tok * _K], i1)
    plsc.store_scatter(idx_v, [tok * _K + 1], i2)


def _make_route(n_tok):
    tpw = n_tok // _NW
    chunks = tpw // _L

    def body(lg_hbm, mask_hbm, w_hbm, idx_hbm, lg_v, mask_v, w_v, idx_v):
        wid = lax.axis_index("s") * _NC + lax.axis_index("c")
        base = wid * tpw
        pltpu.sync_copy(lg_hbm.at[:, pl.ds(base, tpw)], lg_v)

        iota = lax.iota(jnp.int32, _L)
        ones_f = jnp.ones((_L,), jnp.float32)

        def chunk(j, carry):
            off = j * _L
            vs = [lg_v[e, pl.ds(off, _L)] for e in range(_E)]
            _route_chunk(vs, off, iota, ones_f, mask_v, w_v, idx_v)
            return carry

        lax.fori_loop(0, chunks, chunk, 0)

        pltpu.sync_copy(mask_v, mask_hbm.at[pl.ds(base * _E, tpw * _E)])
        pltpu.sync_copy(w_v, w_hbm.at[pl.ds(base * _K, tpw * _K)])
        pltpu.sync_copy(idx_v, idx_hbm.at[pl.ds(base * _K, tpw * _K)])

    mesh = plsc.VectorSubcoreMesh(core_axis_name="c", subcore_axis_name="s")
    return pl.kernel(
        body,
        mesh=mesh,
        compiler_params=pltpu.CompilerParams(needs_layout_passes=False),
        out_type=[
            jax.ShapeDtypeStruct((n_tok * _E,), jnp.float32),
            jax.ShapeDtypeStruct((n_tok * _K,), jnp.float32),
            jax.ShapeDtypeStruct((n_tok * _K,), jnp.int32),
        ],
        scratch_types=[
            pltpu.VMEM((_E, tpw), jnp.float32),
            pltpu.VMEM((tpw * _E,), jnp.float32),
            pltpu.VMEM((tpw * _K,), jnp.float32),
            pltpu.VMEM((tpw * _K,), jnp.int32),
        ],
    )


def kernel(x, gate_w):
    logits_t = _compute_logits(x, gate_w)
    mask, w, idx = _make_route(_N)(logits_t)
    return (
        mask.reshape(_N, _E),
        w.reshape(_N, _K),
        idx.reshape(_N, _K),
    )
